# baseline (speedup 1.0000x reference)
import functools

import jax
import jax.numpy as jnp
from jax.experimental import pallas as pl
from jax.experimental.pallas import tpu as pltpu

B, T, H = 32, 16, 768
E, K, F = 64, 8, 256
N = B * T
EPB = 2
_NEG = -3.0e38


def _body(x_ref, rw_ref, wg_ref, wu_ref, wd_ref, out_ref, logits_ref,
          xb_ref, comb_ref):
    i = pl.program_id(0)

    @pl.when(i == 0)
    def _router():
        x = x_ref[...]
        logits = jnp.dot(x, rw_ref[...], preferred_element_type=jnp.float32)
        logits_ref[...] = logits
        m = jnp.max(logits, axis=1, keepdims=True)
        ex = jnp.exp(logits - m)
        probs = ex / jnp.sum(ex, axis=1, keepdims=True)
        col = jax.lax.broadcasted_iota(jnp.int32, (N, E), 1)
        remaining = probs
        picked = jnp.zeros((N, E), dtype=jnp.bool_)
        for _ in range(K):
            mk = jnp.max(remaining, axis=1, keepdims=True)
            is_max = remaining == mk
            first = jnp.min(jnp.where(is_max, col, E), axis=1, keepdims=True)
            sel = col == first
            picked = jnp.logical_or(picked, sel)
            remaining = jnp.where(sel, _NEG, remaining)
        topk = jnp.where(picked, probs, 0.0)
        comb_ref[...] = topk / jnp.sum(topk, axis=1, keepdims=True)
        xb_ref[...] = x.astype(jnp.bfloat16)

    xb = xb_ref[...]
    acc = None
    for j in range(EPB):
        wg = wg_ref[j].astype(jnp.bfloat16)
        wu = wu_ref[j].astype(jnp.bfloat16)
        g = jnp.dot(xb, wg, preferred_element_type=jnp.float32)
        u = jnp.dot(xb, wu, preferred_element_type=jnp.float32)
        a = (g * jax.nn.sigmoid(g)) * u
        col = jax.lax.broadcasted_iota(jnp.int32, (N, E), 1)
        c = jnp.sum(jnp.where(col == i * EPB + j, comb_ref[...], 0.0),
                    axis=1, keepdims=True)
        wd = wd_ref[j].astype(jnp.bfloat16)
        y = jnp.dot((c * a).astype(jnp.bfloat16), wd,
                    preferred_element_type=jnp.float32)
        acc = y if acc is None else acc + y

    @pl.when(i == 0)
    def _():
        out_ref[...] = acc

    @pl.when(i != 0)
    def _():
        out_ref[...] += acc


@functools.partial(jax.jit, static_argnames=())
def kernel(x, router_w, w_gate, w_up, w_down):
    flat = x.reshape(N, H)
    out, logits = pl.pallas_call(
        _body,
        grid=(E // EPB,),
        in_specs=[
            pl.BlockSpec((N, H), lambda i: (0, 0)),
            pl.BlockSpec((H, E), lambda i: (0, 0)),
            pl.BlockSpec((EPB, H, F), lambda i: (i, 0, 0)),
            pl.BlockSpec((EPB, H, F), lambda i: (i, 0, 0)),
            pl.BlockSpec((EPB, F, H), lambda i: (i, 0, 0)),
        ],
        out_specs=(
            pl.BlockSpec((N, H), lambda i: (0, 0)),
            pl.BlockSpec((N, E), lambda i: (0, 0)),
        ),
        out_shape=(
            jax.ShapeDtypeStruct((N, H), jnp.float32),
            jax.ShapeDtypeStruct((N, E), jnp.float32),
        ),
        scratch_shapes=[
            pltpu.VMEM((N, H), jnp.bfloat16),
            pltpu.VMEM((N, E), jnp.float32),
        ],
    )(flat, router_w, w_gate, w_up, w_down)

    return out.reshape(B, T, H), logits


# speedup vs baseline: 1.1128x; 1.1128x over previous
import functools

import jax
import jax.numpy as jnp
from jax.experimental import pallas as pl
from jax.experimental.pallas import tpu as pltpu

B, T, H = 32, 16, 768
E, K, F = 64, 8, 256
N = B * T
EPB = 4
_NEG = -3.0e38


def _body(x_ref, rw_ref, wg_ref, wu_ref, wd_ref, out_ref, logits_ref,
          xb_ref, comb_ref):
    i = pl.program_id(0)

    @pl.when(i == 0)
    def _router():
        x = x_ref[...]
        logits = jnp.dot(x, rw_ref[...], preferred_element_type=jnp.float32)
        logits_ref[...] = logits
        m = jnp.max(logits, axis=1, keepdims=True)
        ex = jnp.exp(logits - m)
        probs = ex / jnp.sum(ex, axis=1, keepdims=True)
        col = jax.lax.broadcasted_iota(jnp.int32, (N, E), 1)
        remaining = probs
        picked = jnp.zeros((N, E), dtype=jnp.bool_)
        for _ in range(K):
            mk = jnp.max(remaining, axis=1, keepdims=True)
            is_max = remaining == mk
            first = jnp.min(jnp.where(is_max, col, E), axis=1, keepdims=True)
            sel = col == first
            picked = jnp.logical_or(picked, sel)
            remaining = jnp.where(sel, _NEG, remaining)
        topk = jnp.where(picked, probs, 0.0)
        comb_ref[...] = topk / jnp.sum(topk, axis=1, keepdims=True)
        xb_ref[...] = x.astype(jnp.bfloat16)

    xb = xb_ref[...]
    acc = None
    for j in range(EPB):
        wg = wg_ref[j].astype(jnp.bfloat16)
        wu = wu_ref[j].astype(jnp.bfloat16)
        g = jnp.dot(xb, wg, preferred_element_type=jnp.float32)
        u = jnp.dot(xb, wu, preferred_element_type=jnp.float32)
        a = (g * jax.nn.sigmoid(g)) * u
        col = jax.lax.broadcasted_iota(jnp.int32, (N, E), 1)
        c = jnp.sum(jnp.where(col == i * EPB + j, comb_ref[...], 0.0),
                    axis=1, keepdims=True)
        wd = wd_ref[j].astype(jnp.bfloat16)
        y = jnp.dot((c * a).astype(jnp.bfloat16), wd,
                    preferred_element_type=jnp.float32)
        acc = y if acc is None else acc + y

    @pl.when(i == 0)
    def _():
        out_ref[...] = acc

    @pl.when(i != 0)
    def _():
        out_ref[...] += acc


@functools.partial(jax.jit, static_argnames=())
def kernel(x, router_w, w_gate, w_up, w_down):
    flat = x.reshape(N, H)
    out, logits = pl.pallas_call(
        _body,
        grid=(E // EPB,),
        in_specs=[
            pl.BlockSpec((N, H), lambda i: (0, 0)),
            pl.BlockSpec((H, E), lambda i: (0, 0)),
            pl.BlockSpec((EPB, H, F), lambda i: (i, 0, 0)),
            pl.BlockSpec((EPB, H, F), lambda i: (i, 0, 0)),
            pl.BlockSpec((EPB, F, H), lambda i: (i, 0, 0)),
        ],
        out_specs=(
            pl.BlockSpec((N, H), lambda i: (0, 0)),
            pl.BlockSpec((N, E), lambda i: (0, 0)),
        ),
        out_shape=(
            jax.ShapeDtypeStruct((N, H), jnp.float32),
            jax.ShapeDtypeStruct((N, E), jnp.float32),
        ),
        scratch_shapes=[
            pltpu.VMEM((N, H), jnp.bfloat16),
            pltpu.VMEM((N, E), jnp.float32),
        ],
    )(flat, router_w, w_gate, w_up, w_down)

    return out.reshape(B, T, H), logits


# speedup vs baseline: 1.1303x; 1.0157x over previous
import functools

import jax
import jax.numpy as jnp
from jax.experimental import pallas as pl
from jax.experimental.pallas import tpu as pltpu

B, T, H = 32, 16, 768
E, K, F = 64, 8, 256
N = B * T
EPB = 8
_NEG = -3.0e38


def _body(x_ref, rw_ref, wg_ref, wu_ref, wd_ref, out_ref, logits_ref,
          xb_ref, comb_ref):
    i = pl.program_id(0)

    @pl.when(i == 0)
    def _router():
        x = x_ref[...]
        logits = jnp.dot(x, rw_ref[...], preferred_element_type=jnp.float32)
        logits_ref[...] = logits
        m = jnp.max(logits, axis=1, keepdims=True)
        ex = jnp.exp(logits - m)
        probs = ex / jnp.sum(ex, axis=1, keepdims=True)
        col = jax.lax.broadcasted_iota(jnp.int32, (N, E), 1)
        remaining = probs
        picked = jnp.zeros((N, E), dtype=jnp.bool_)
        for _ in range(K):
            mk = jnp.max(remaining, axis=1, keepdims=True)
            is_max = remaining == mk
            first = jnp.min(jnp.where(is_max, col, E), axis=1, keepdims=True)
            sel = col == first
            picked = jnp.logical_or(picked, sel)
            remaining = jnp.where(sel, _NEG, remaining)
        topk = jnp.where(picked, probs, 0.0)
        comb_ref[...] = topk / jnp.sum(topk, axis=1, keepdims=True)
        xb_ref[...] = x.astype(jnp.bfloat16)

    xb = xb_ref[...]
    acc = None
    for j in range(EPB):
        wg = wg_ref[j].astype(jnp.bfloat16)
        wu = wu_ref[j].astype(jnp.bfloat16)
        g = jnp.dot(xb, wg, preferred_element_type=jnp.float32)
        u = jnp.dot(xb, wu, preferred_element_type=jnp.float32)
        a = (g * jax.nn.sigmoid(g)) * u
        col = jax.lax.broadcasted_iota(jnp.int32, (N, E), 1)
        c = jnp.sum(jnp.where(col == i * EPB + j, comb_ref[...], 0.0),
                    axis=1, keepdims=True)
        wd = wd_ref[j].astype(jnp.bfloat16)
        y = jnp.dot((c * a).astype(jnp.bfloat16), wd,
                    preferred_element_type=jnp.float32)
        acc = y if acc is None else acc + y

    @pl.when(i == 0)
    def _():
        out_ref[...] = acc

    @pl.when(i != 0)
    def _():
        out_ref[...] += acc


@functools.partial(jax.jit, static_argnames=())
def kernel(x, router_w, w_gate, w_up, w_down):
    flat = x.reshape(N, H)
    out, logits = pl.pallas_call(
        _body,
        grid=(E // EPB,),
        in_specs=[
            pl.BlockSpec((N, H), lambda i: (0, 0)),
            pl.BlockSpec((H, E), lambda i: (0, 0)),
            pl.BlockSpec((EPB, H, F), lambda i: (i, 0, 0)),
            pl.BlockSpec((EPB, H, F), lambda i: (i, 0, 0)),
            pl.BlockSpec((EPB, F, H), lambda i: (i, 0, 0)),
        ],
        out_specs=(
            pl.BlockSpec((N, H), lambda i: (0, 0)),
            pl.BlockSpec((N, E), lambda i: (0, 0)),
        ),
        out_shape=(
            jax.ShapeDtypeStruct((N, H), jnp.float32),
            jax.ShapeDtypeStruct((N, E), jnp.float32),
        ),
        scratch_shapes=[
            pltpu.VMEM((N, H), jnp.bfloat16),
            pltpu.VMEM((N, E), jnp.float32),
        ],
    )(flat, router_w, w_gate, w_up, w_down)

    return out.reshape(B, T, H), logits
